# Initial kernel scaffold; baseline (speedup 1.0000x reference)
import jax
import jax.numpy as jnp
from jax.experimental import pallas as pl


def kernel(x, edge_index, W, b):
    raise NotImplementedError("write your pallas kernel here")


# speedup vs baseline: 13.4589x; 13.4589x over previous
import functools

import jax
import jax.numpy as jnp
from jax import lax
from jax.experimental import pallas as pl
from jax.experimental.pallas import tpu as pltpu
from jax.experimental.pallas import tpu_sc as plsc

N = 10000
E = 320000
D = 128

NC, NS = 2, 16
NW = NC * NS
CHUNK = 128
NCHUNK = 80
EPW = NCHUNK * CHUNK
E_PAD = EPW * NW
NPAD = 10240
RPW = NPAD // NS
GC = 40
NGRP = NCHUNK // GC

_mesh = plsc.VectorSubcoreMesh(
    core_axis_name="c", subcore_axis_name="s", num_cores=NC, num_subcores=NS
)


WD = 128


@functools.partial(
    pl.kernel,
    out_type=jax.ShapeDtypeStruct((NC, NPAD, WD), jnp.float32),
    mesh=_mesh,
    scratch_types=[
        pltpu.VMEM((NCHUNK, CHUNK), jnp.int32),
        pltpu.VMEM((CHUNK, WD), jnp.float32),
        pltpu.VMEM_SHARED((NPAD, WD), jnp.float32),
    ],
)
def _deg_kernel(dst_hbm, zeros_hbm, ones_hbm, acc_out, idx_v, ones_v, acc_sh):
    cid = lax.axis_index("c")
    sid = lax.axis_index("s")
    wid = cid * NS + sid

    pltpu.sync_copy(ones_hbm, ones_v)
    pltpu.sync_copy(
        zeros_hbm.at[pl.ds(sid * RPW, RPW)], acc_sh.at[pl.ds(sid * RPW, RPW)]
    )
    pltpu.sync_copy(dst_hbm.at[wid], idx_v)
    plsc.subcore_barrier()

    def body(j, carry):
        pltpu.sync_copy(ones_v, acc_sh.at[idx_v.at[j]], add=True)
        return carry

    lax.fori_loop(0, NCHUNK, body, 0)
    plsc.subcore_barrier()
    pltpu.sync_copy(
        acc_sh.at[pl.ds(sid * RPW, RPW)], acc_out.at[cid, pl.ds(sid * RPW, RPW)]
    )


@functools.partial(
    pl.kernel,
    out_type=jax.ShapeDtypeStruct((NC, NPAD, D), jnp.float32),
    mesh=_mesh,
    scratch_types=[
        pltpu.VMEM((GC, CHUNK), jnp.int32),
        pltpu.VMEM((GC, CHUNK), jnp.int32),
        pltpu.VMEM((CHUNK, D), jnp.float32),
        pltpu.VMEM((CHUNK, D), jnp.float32),
        pltpu.VMEM_SHARED((NPAD, D), jnp.float32),
        pltpu.SemaphoreType.DMA,
        pltpu.SemaphoreType.DMA,
    ],
)
def _agg_kernel(
    src_hbm, dst_hbm, hs_hbm, zeros_hbm, part_out,
    src_v, dst_v, buf0, buf1, acc_sh, sem0, sem1,
):
    cid = lax.axis_index("c")
    sid = lax.axis_index("s")
    wid = cid * NS + sid

    pltpu.sync_copy(
        zeros_hbm.at[pl.ds(sid * RPW, RPW)], acc_sh.at[pl.ds(sid * RPW, RPW)]
    )
    plsc.subcore_barrier()

    def group(g, carry):
        pltpu.sync_copy(src_hbm.at[wid, pl.ds(g * GC, GC)], src_v)
        pltpu.sync_copy(dst_hbm.at[wid, pl.ds(g * GC, GC)], dst_v)
        pltpu.async_copy(hs_hbm.at[src_v.at[0]], buf0, sem0)

        def body(jj, carry2):
            j = jj * 2

            pltpu.async_copy(hs_hbm.at[src_v.at[j + 1]], buf1, sem1)
            pltpu.make_async_copy(hs_hbm.at[src_v.at[j]], buf0, sem0).wait()
            pltpu.sync_copy(buf0, acc_sh.at[dst_v.at[j]], add=True)

            @pl.when(j + 2 < GC)
            def _():
                pltpu.async_copy(hs_hbm.at[src_v.at[j + 2]], buf0, sem0)

            pltpu.make_async_copy(hs_hbm.at[src_v.at[j + 1]], buf1, sem1).wait()
            pltpu.sync_copy(buf1, acc_sh.at[dst_v.at[j + 1]], add=True)
            return carry2

        lax.fori_loop(0, GC // 2, body, 0)
        return carry

    lax.fori_loop(0, NGRP, group, 0)
    plsc.subcore_barrier()
    pltpu.sync_copy(
        acc_sh.at[pl.ds(sid * RPW, RPW)], part_out.at[cid, pl.ds(sid * RPW, RPW)]
    )


_RB = 2000


def _hs_body(x_ref, w_ref, dega_ref, hs_ref):
    h = jnp.dot(x_ref[...], w_ref[...], preferred_element_type=jnp.float32)
    deg = dega_ref[0, :, 0] + dega_ref[1, :, 0] + 1.0
    dis = lax.rsqrt(deg)
    hs_ref[...] = h * dis[:, None]


def _out_body(p_ref, hs_ref, dega_ref, b_ref, o_ref):
    deg = dega_ref[0, :, 0] + dega_ref[1, :, 0] + 1.0
    dis = lax.rsqrt(deg)
    s = p_ref[0] + p_ref[1] + hs_ref[...]
    o_ref[...] = s * dis[:, None] + b_ref[...]


def kernel(x, edge_index, W, b):
    src = edge_index[0].astype(jnp.int32)
    dst = edge_index[1].astype(jnp.int32)
    pad = E_PAD - E
    src_p = jnp.concatenate([src, jnp.zeros((pad,), jnp.int32)]).reshape(
        NW, NCHUNK, CHUNK
    )
    dst_p = jnp.concatenate([dst, jnp.full((pad,), N, jnp.int32)]).reshape(
        NW, NCHUNK, CHUNK
    )
    zerosW = jnp.zeros((NPAD, WD), jnp.float32)
    zerosD = jnp.zeros((NPAD, D), jnp.float32)
    onesW = jnp.ones((CHUNK, WD), jnp.float32)

    dega = _deg_kernel(dst_p, zerosW, onesW)

    hs = pl.pallas_call(
        _hs_body,
        grid=(N // _RB,),
        in_specs=[
            pl.BlockSpec((_RB, D), lambda i: (i, 0)),
            pl.BlockSpec((D, D), lambda i: (0, 0)),
            pl.BlockSpec((NC, _RB, WD), lambda i: (0, i, 0)),
        ],
        out_specs=pl.BlockSpec((_RB, D), lambda i: (i, 0)),
        out_shape=jax.ShapeDtypeStruct((N, D), jnp.float32),
    )(x, W, dega)

    parts = _agg_kernel(src_p, dst_p, hs, zerosD)

    out = pl.pallas_call(
        _out_body,
        grid=(N // _RB,),
        in_specs=[
            pl.BlockSpec((NC, _RB, D), lambda i: (0, i, 0)),
            pl.BlockSpec((_RB, D), lambda i: (i, 0)),
            pl.BlockSpec((NC, _RB, WD), lambda i: (0, i, 0)),
            pl.BlockSpec((1, D), lambda i: (0, 0)),
        ],
        out_specs=pl.BlockSpec((_RB, D), lambda i: (i, 0)),
        out_shape=jax.ShapeDtypeStruct((N, D), jnp.float32),
    )(parts, hs, dega, b.reshape(1, D))
    return out


# speedup vs baseline: 32.2824x; 2.3986x over previous
import functools

import jax
import jax.numpy as jnp
from jax import lax
from jax.experimental import pallas as pl
from jax.experimental.pallas import tpu as pltpu
from jax.experimental.pallas import tpu_sc as plsc

N = 10000
E = 320000
D = 128

NC, NS = 2, 16
NW = NC * NS
CHUNK = 128
NCHUNK = 80
EPW = NCHUNK * CHUNK
E_PAD = EPW * NW
NPAD = 10240
RPW = NPAD // NS
GC = 40
NGRP = NCHUNK // GC

_mesh = plsc.VectorSubcoreMesh(
    core_axis_name="c", subcore_axis_name="s", num_cores=NC, num_subcores=NS
)


WD = 128


@functools.partial(
    pl.kernel,
    out_type=jax.ShapeDtypeStruct((NC, NPAD, WD), jnp.float32),
    mesh=_mesh,
    scratch_types=[
        pltpu.VMEM((NCHUNK, CHUNK), jnp.int32),
        pltpu.VMEM((CHUNK, WD), jnp.float32),
        pltpu.VMEM_SHARED((NPAD, WD), jnp.float32),
    ],
)
def _deg_kernel(dst_hbm, zeros_hbm, ones_hbm, acc_out, idx_v, ones_v, acc_sh):
    cid = lax.axis_index("c")
    sid = lax.axis_index("s")
    wid = cid * NS + sid

    pltpu.sync_copy(ones_hbm, ones_v)
    pltpu.sync_copy(
        zeros_hbm.at[pl.ds(sid * RPW, RPW)], acc_sh.at[pl.ds(sid * RPW, RPW)]
    )
    pltpu.sync_copy(dst_hbm.at[wid], idx_v)
    plsc.subcore_barrier()

    def body(j, carry):
        pltpu.sync_copy(ones_v, acc_sh.at[idx_v.at[j]], add=True)
        return carry

    lax.fori_loop(0, NCHUNK, body, 0)
    plsc.subcore_barrier()
    pltpu.sync_copy(
        acc_sh.at[pl.ds(sid * RPW, RPW)], acc_out.at[cid, pl.ds(sid * RPW, RPW)]
    )


@functools.partial(
    pl.kernel,
    out_type=jax.ShapeDtypeStruct((NC, NPAD, D), jnp.float32),
    mesh=_mesh,
    scratch_types=[
        pltpu.VMEM((GC, CHUNK), jnp.int32),
        pltpu.VMEM((GC, CHUNK), jnp.int32),
        pltpu.VMEM((CHUNK, D), jnp.float32),
        pltpu.VMEM((CHUNK, D), jnp.float32),
        pltpu.VMEM_SHARED((NPAD, D), jnp.float32),
        pltpu.SemaphoreType.DMA,
        pltpu.SemaphoreType.DMA,
    ],
)
def _agg_kernel(
    src_hbm, dst_hbm, hs_hbm, zeros_hbm, part_out,
    src_v, dst_v, buf0, buf1, acc_sh, sem0, sem1,
):
    cid = lax.axis_index("c")
    sid = lax.axis_index("s")
    wid = cid * NS + sid

    pltpu.sync_copy(
        zeros_hbm.at[pl.ds(sid * RPW, RPW)], acc_sh.at[pl.ds(sid * RPW, RPW)]
    )
    plsc.subcore_barrier()

    def group(g, carry):
        pltpu.sync_copy(src_hbm.at[wid, pl.ds(g * GC, GC)], src_v)
        pltpu.sync_copy(dst_hbm.at[wid, pl.ds(g * GC, GC)], dst_v)
        pltpu.async_copy(hs_hbm.at[src_v.at[0]], buf0, sem0)

        def body(jj, carry2):
            j = jj * 2

            pltpu.async_copy(hs_hbm.at[src_v.at[j + 1]], buf1, sem1)
            pltpu.make_async_copy(hs_hbm.at[src_v.at[j]], buf0, sem0).wait()
            pltpu.sync_copy(buf0, acc_sh.at[dst_v.at[j]], add=True)

            @pl.when(j + 2 < GC)
            def _():
                pltpu.async_copy(hs_hbm.at[src_v.at[j + 2]], buf0, sem0)

            pltpu.make_async_copy(hs_hbm.at[src_v.at[j + 1]], buf1, sem1).wait()
            pltpu.sync_copy(buf1, acc_sh.at[dst_v.at[j + 1]], add=True)
            return carry2

        lax.fori_loop(0, GC // 2, body, 0)
        return carry

    lax.fori_loop(0, NGRP, group, 0)
    plsc.subcore_barrier()
    pltpu.sync_copy(
        acc_sh.at[pl.ds(sid * RPW, RPW)], part_out.at[cid, pl.ds(sid * RPW, RPW)]
    )


_RB = 2000


def _hs_body(x_ref, w_ref, dega_ref, hs_ref):
    h = jnp.dot(x_ref[...], w_ref[...], preferred_element_type=jnp.float32)
    deg = dega_ref[0, :, 0] + dega_ref[1, :, 0] + 1.0
    dis = lax.rsqrt(deg)
    hs_ref[...] = h * dis[:, None]


def _out_body(p_ref, hs_ref, dega_ref, b_ref, o_ref):
    deg = dega_ref[0, :, 0] + dega_ref[1, :, 0] + 1.0
    dis = lax.rsqrt(deg)
    s = p_ref[0] + p_ref[1] + hs_ref[...]
    o_ref[...] = s * dis[:, None] + b_ref[...]


def kernel(x, edge_index, W, b):
    src = edge_index[0].astype(jnp.int32)
    dst = edge_index[1].astype(jnp.int32)
    pad = E_PAD - E
    pad_idx = jnp.arange(pad, dtype=jnp.int32)
    src_p = jnp.concatenate([src, pad_idx % N]).reshape(NW, NCHUNK, CHUNK)
    dst_p = jnp.concatenate([dst, N + pad_idx % (NPAD - N)]).reshape(
        NW, NCHUNK, CHUNK
    )
    zerosW = jnp.zeros((NPAD, WD), jnp.float32)
    zerosD = jnp.zeros((NPAD, D), jnp.float32)
    onesW = jnp.ones((CHUNK, WD), jnp.float32)

    dega = _deg_kernel(dst_p, zerosW, onesW)

    hs = pl.pallas_call(
        _hs_body,
        grid=(N // _RB,),
        in_specs=[
            pl.BlockSpec((_RB, D), lambda i: (i, 0)),
            pl.BlockSpec((D, D), lambda i: (0, 0)),
            pl.BlockSpec((NC, _RB, WD), lambda i: (0, i, 0)),
        ],
        out_specs=pl.BlockSpec((_RB, D), lambda i: (i, 0)),
        out_shape=jax.ShapeDtypeStruct((N, D), jnp.float32),
    )(x, W, dega)

    parts = _agg_kernel(src_p, dst_p, hs, zerosD)

    out = pl.pallas_call(
        _out_body,
        grid=(N // _RB,),
        in_specs=[
            pl.BlockSpec((NC, _RB, D), lambda i: (0, i, 0)),
            pl.BlockSpec((_RB, D), lambda i: (i, 0)),
            pl.BlockSpec((NC, _RB, WD), lambda i: (0, i, 0)),
            pl.BlockSpec((1, D), lambda i: (0, 0)),
        ],
        out_specs=pl.BlockSpec((_RB, D), lambda i: (i, 0)),
        out_shape=jax.ShapeDtypeStruct((N, D), jnp.float32),
    )(parts, hs, dega, b.reshape(1, D))
    return out


# speedup vs baseline: 33.0292x; 1.0231x over previous
import functools

import jax
import jax.numpy as jnp
from jax import lax
from jax.experimental import pallas as pl
from jax.experimental.pallas import tpu as pltpu
from jax.experimental.pallas import tpu_sc as plsc

N = 10000
E = 320000
D = 128

NC, NS = 2, 16
NW = NC * NS
CHUNK = 128
NCHUNK = 80
EPW = NCHUNK * CHUNK
E_PAD = EPW * NW
NPAD = 10240
RPW = NPAD // NS
CHA = 64
NCHA = EPW // CHA
GC = 40
NGRP = NCHA // GC

_mesh = plsc.VectorSubcoreMesh(
    core_axis_name="c", subcore_axis_name="s", num_cores=NC, num_subcores=NS
)


WD = 128


@functools.partial(
    pl.kernel,
    out_type=jax.ShapeDtypeStruct((NC, NPAD, WD), jnp.float32),
    mesh=_mesh,
    scratch_types=[
        pltpu.VMEM((NCHUNK, CHUNK), jnp.int32),
        pltpu.VMEM((CHUNK, WD), jnp.float32),
        pltpu.VMEM_SHARED((NPAD, WD), jnp.float32),
    ],
)
def _deg_kernel(dst_hbm, zeros_hbm, ones_hbm, acc_out, idx_v, ones_v, acc_sh):
    cid = lax.axis_index("c")
    sid = lax.axis_index("s")
    wid = cid * NS + sid

    pltpu.sync_copy(ones_hbm, ones_v)
    pltpu.sync_copy(
        zeros_hbm.at[pl.ds(sid * RPW, RPW)], acc_sh.at[pl.ds(sid * RPW, RPW)]
    )
    pltpu.sync_copy(dst_hbm.at[wid], idx_v)
    plsc.subcore_barrier()

    def body(j, carry):
        pltpu.sync_copy(ones_v, acc_sh.at[idx_v.at[j]], add=True)
        return carry

    lax.fori_loop(0, NCHUNK, body, 0)
    plsc.subcore_barrier()
    pltpu.sync_copy(
        acc_sh.at[pl.ds(sid * RPW, RPW)], acc_out.at[cid, pl.ds(sid * RPW, RPW)]
    )


@functools.partial(
    pl.kernel,
    out_type=jax.ShapeDtypeStruct((NC, NPAD, D), jnp.float32),
    mesh=_mesh,
    scratch_types=[
        pltpu.VMEM((GC, CHA), jnp.int32),
        pltpu.VMEM((GC, CHA), jnp.int32),
        [pltpu.VMEM((CHA, D), jnp.float32) for _ in range(4)],
        [pltpu.SemaphoreType.DMA for _ in range(4)],
        pltpu.VMEM_SHARED((NPAD, D), jnp.float32),
    ],
)
def _agg_kernel(
    src_hbm, dst_hbm, hs_hbm, zeros_hbm, part_out,
    src_v, dst_v, bufs, sems, acc_sh,
):
    cid = lax.axis_index("c")
    sid = lax.axis_index("s")
    wid = cid * NS + sid

    pltpu.sync_copy(
        zeros_hbm.at[pl.ds(sid * RPW, RPW)], acc_sh.at[pl.ds(sid * RPW, RPW)]
    )
    plsc.subcore_barrier()

    def group(g, carry):
        pltpu.sync_copy(src_hbm.at[wid, pl.ds(g * GC, GC)], src_v)
        pltpu.sync_copy(dst_hbm.at[wid, pl.ds(g * GC, GC)], dst_v)
        for b in range(3):
            pltpu.async_copy(hs_hbm.at[src_v.at[b]], bufs[b], sems[b])

        def body(jj, carry2):
            for b4 in range(4):
                j = jj * 4 + b4
                pltpu.make_async_copy(
                    hs_hbm.at[src_v.at[j]], bufs[b4], sems[b4]
                ).wait()

                @pl.when(j + 3 < GC)
                def _():
                    pltpu.async_copy(
                        hs_hbm.at[src_v.at[j + 3]],
                        bufs[(b4 + 3) % 4],
                        sems[(b4 + 3) % 4],
                    )

                pltpu.sync_copy(bufs[b4], acc_sh.at[dst_v.at[j]], add=True)
            return carry2

        lax.fori_loop(0, GC // 4, body, 0)
        return carry

    lax.fori_loop(0, NGRP, group, 0)
    plsc.subcore_barrier()
    pltpu.sync_copy(
        acc_sh.at[pl.ds(sid * RPW, RPW)], part_out.at[cid, pl.ds(sid * RPW, RPW)]
    )


_RB = 2000


def _hs_body(x_ref, w_ref, dega_ref, hs_ref):
    h = jnp.dot(x_ref[...], w_ref[...], preferred_element_type=jnp.float32)
    deg = dega_ref[0, :, 0] + dega_ref[1, :, 0] + 1.0
    dis = lax.rsqrt(deg)
    hs_ref[...] = h * dis[:, None]


def _out_body(p_ref, hs_ref, dega_ref, b_ref, o_ref):
    deg = dega_ref[0, :, 0] + dega_ref[1, :, 0] + 1.0
    dis = lax.rsqrt(deg)
    s = p_ref[0] + p_ref[1] + hs_ref[...]
    o_ref[...] = s * dis[:, None] + b_ref[...]


def kernel(x, edge_index, W, b):
    src = edge_index[0].astype(jnp.int32)
    dst = edge_index[1].astype(jnp.int32)
    pad = E_PAD - E
    pad_idx = jnp.arange(pad, dtype=jnp.int32)
    src_p = jnp.concatenate([src, pad_idx % N]).reshape(NW, NCHUNK, CHUNK)
    dst_p = jnp.concatenate([dst, N + pad_idx % (NPAD - N)]).reshape(
        NW, NCHUNK, CHUNK
    )
    zerosW = jnp.zeros((NPAD, WD), jnp.float32)
    zerosD = jnp.zeros((NPAD, D), jnp.float32)
    onesW = jnp.ones((CHUNK, WD), jnp.float32)

    dega = _deg_kernel(dst_p, zerosW, onesW)

    hs = pl.pallas_call(
        _hs_body,
        grid=(N // _RB,),
        in_specs=[
            pl.BlockSpec((_RB, D), lambda i: (i, 0)),
            pl.BlockSpec((D, D), lambda i: (0, 0)),
            pl.BlockSpec((NC, _RB, WD), lambda i: (0, i, 0)),
        ],
        out_specs=pl.BlockSpec((_RB, D), lambda i: (i, 0)),
        out_shape=jax.ShapeDtypeStruct((N, D), jnp.float32),
    )(x, W, dega)

    parts = _agg_kernel(
        src_p.reshape(NW, NCHA, CHA), dst_p.reshape(NW, NCHA, CHA), hs, zerosD
    )

    out = pl.pallas_call(
        _out_body,
        grid=(N // _RB,),
        in_specs=[
            pl.BlockSpec((NC, _RB, D), lambda i: (0, i, 0)),
            pl.BlockSpec((_RB, D), lambda i: (i, 0)),
            pl.BlockSpec((NC, _RB, WD), lambda i: (0, i, 0)),
            pl.BlockSpec((1, D), lambda i: (0, 0)),
        ],
        out_specs=pl.BlockSpec((_RB, D), lambda i: (i, 0)),
        out_shape=jax.ShapeDtypeStruct((N, D), jnp.float32),
    )(parts, hs, dega, b.reshape(1, D))
    return out


# speedup vs baseline: 33.2862x; 1.0078x over previous
import functools

import jax
import jax.numpy as jnp
from jax import lax
from jax.experimental import pallas as pl
from jax.experimental.pallas import tpu as pltpu
from jax.experimental.pallas import tpu_sc as plsc

N = 10000
E = 320000
D = 128

NC, NS = 2, 16
NW = NC * NS
CHUNK = 128
NCHUNK = 80
EPW = NCHUNK * CHUNK
E_PAD = EPW * NW
NPAD = 10240
RPW = NPAD // NS
CHA = 64
NCHA = EPW // CHA
GC = 40
NGRP = NCHA // GC

_mesh = plsc.VectorSubcoreMesh(
    core_axis_name="c", subcore_axis_name="s", num_cores=NC, num_subcores=NS
)


WD = 128


@functools.partial(
    pl.kernel,
    out_type=jax.ShapeDtypeStruct((NC, NPAD, WD), jnp.float32),
    mesh=_mesh,
    scratch_types=[
        pltpu.VMEM((NCHUNK, CHUNK), jnp.int32),
        pltpu.VMEM((CHUNK, WD), jnp.float32),
        [pltpu.SemaphoreType.DMA for _ in range(2)],
        pltpu.VMEM_SHARED((NPAD, WD), jnp.float32),
    ],
)
def _deg_kernel(dst_hbm, zeros_hbm, ones_hbm, acc_out, idx_v, ones_v, sems, acc_sh):
    cid = lax.axis_index("c")
    sid = lax.axis_index("s")
    wid = cid * NS + sid

    pltpu.sync_copy(ones_hbm, ones_v)
    pltpu.sync_copy(
        zeros_hbm.at[pl.ds(sid * RPW, RPW)], acc_sh.at[pl.ds(sid * RPW, RPW)]
    )
    pltpu.sync_copy(dst_hbm.at[wid], idx_v)
    plsc.subcore_barrier()

    def body(jj, carry):
        j = jj * 2
        pltpu.async_copy(ones_v, acc_sh.at[idx_v.at[j + 1]], sems[1], add=True)
        pltpu.make_async_copy(ones_v, acc_sh.at[idx_v.at[j]], sems[0]).wait()

        @pl.when(j + 2 < NCHUNK)
        def _():
            pltpu.async_copy(ones_v, acc_sh.at[idx_v.at[j + 2]], sems[0], add=True)

        pltpu.make_async_copy(ones_v, acc_sh.at[idx_v.at[j + 1]], sems[1]).wait()
        return carry

    pltpu.async_copy(ones_v, acc_sh.at[idx_v.at[0]], sems[0], add=True)
    lax.fori_loop(0, NCHUNK // 2, body, 0)
    plsc.subcore_barrier()
    pltpu.sync_copy(
        acc_sh.at[pl.ds(sid * RPW, RPW)], acc_out.at[cid, pl.ds(sid * RPW, RPW)]
    )


@functools.partial(
    pl.kernel,
    out_type=jax.ShapeDtypeStruct((NC, NPAD, D), jnp.float32),
    mesh=_mesh,
    scratch_types=[
        pltpu.VMEM((GC, CHA), jnp.int32),
        pltpu.VMEM((GC, CHA), jnp.int32),
        [pltpu.VMEM((CHA, D), jnp.float32) for _ in range(4)],
        [pltpu.SemaphoreType.DMA for _ in range(4)],
        [pltpu.SemaphoreType.DMA for _ in range(4)],
        pltpu.VMEM_SHARED((NPAD, D), jnp.float32),
    ],
)
def _agg_kernel(
    src_hbm, dst_hbm, hs_hbm, zeros_hbm, part_out,
    src_v, dst_v, bufs, sems, ssems, acc_sh,
):
    cid = lax.axis_index("c")
    sid = lax.axis_index("s")
    wid = cid * NS + sid

    pltpu.sync_copy(
        zeros_hbm.at[pl.ds(sid * RPW, RPW)], acc_sh.at[pl.ds(sid * RPW, RPW)]
    )
    plsc.subcore_barrier()

    def group(g, carry):
        pltpu.sync_copy(src_hbm.at[wid, pl.ds(g * GC, GC)], src_v)
        pltpu.sync_copy(dst_hbm.at[wid, pl.ds(g * GC, GC)], dst_v)
        for b in range(3):
            pltpu.async_copy(hs_hbm.at[src_v.at[b]], bufs[b], sems[b])

        def body(jj, carry2):
            for b4 in range(4):
                j = jj * 4 + b4
                bn = (b4 + 3) % 4
                pltpu.make_async_copy(
                    hs_hbm.at[src_v.at[j]], bufs[b4], sems[b4]
                ).wait()

                @pl.when(jnp.logical_and(j + 3 < GC, j >= 1))
                def _():
                    pltpu.make_async_copy(
                        bufs[bn], acc_sh.at[dst_v.at[0]], ssems[bn]
                    ).wait()

                @pl.when(j + 3 < GC)
                def _():
                    pltpu.async_copy(
                        hs_hbm.at[src_v.at[j + 3]], bufs[bn], sems[bn]
                    )

                pltpu.async_copy(
                    bufs[b4], acc_sh.at[dst_v.at[j]], ssems[b4], add=True
                )
            return carry2

        lax.fori_loop(0, GC // 4, body, 0)
        for b in range(4):
            pltpu.make_async_copy(bufs[b], acc_sh.at[dst_v.at[0]], ssems[b]).wait()
        return carry

    lax.fori_loop(0, NGRP, group, 0)
    plsc.subcore_barrier()
    pltpu.sync_copy(
        acc_sh.at[pl.ds(sid * RPW, RPW)], part_out.at[cid, pl.ds(sid * RPW, RPW)]
    )


_RB = 2000


def _hs_body(x_ref, w_ref, dega_ref, hs_ref):
    h = jnp.dot(x_ref[...], w_ref[...], preferred_element_type=jnp.float32)
    deg = dega_ref[0, :, 0] + dega_ref[1, :, 0] + 1.0
    dis = lax.rsqrt(deg)
    hs_ref[...] = h * dis[:, None]


def _out_body(p_ref, hs_ref, dega_ref, b_ref, o_ref):
    deg = dega_ref[0, :, 0] + dega_ref[1, :, 0] + 1.0
    dis = lax.rsqrt(deg)
    s = p_ref[0] + p_ref[1] + hs_ref[...]
    o_ref[...] = s * dis[:, None] + b_ref[...]


def kernel(x, edge_index, W, b):
    src = edge_index[0].astype(jnp.int32)
    dst = edge_index[1].astype(jnp.int32)
    pad = E_PAD - E
    pad_idx = jnp.arange(pad, dtype=jnp.int32)
    src_p = jnp.concatenate([src, pad_idx % N]).reshape(NW, NCHUNK, CHUNK)
    dst_p = jnp.concatenate([dst, N + pad_idx % (NPAD - N)]).reshape(
        NW, NCHUNK, CHUNK
    )
    zerosW = jnp.zeros((NPAD, WD), jnp.float32)
    zerosD = jnp.zeros((NPAD, D), jnp.float32)
    onesW = jnp.ones((CHUNK, WD), jnp.float32)

    dega = _deg_kernel(dst_p, zerosW, onesW)

    hs = pl.pallas_call(
        _hs_body,
        grid=(N // _RB,),
        in_specs=[
            pl.BlockSpec((_RB, D), lambda i: (i, 0)),
            pl.BlockSpec((D, D), lambda i: (0, 0)),
            pl.BlockSpec((NC, _RB, WD), lambda i: (0, i, 0)),
        ],
        out_specs=pl.BlockSpec((_RB, D), lambda i: (i, 0)),
        out_shape=jax.ShapeDtypeStruct((N, D), jnp.float32),
    )(x, W, dega)

    parts = _agg_kernel(
        src_p.reshape(NW, NCHA, CHA), dst_p.reshape(NW, NCHA, CHA), hs, zerosD
    )

    out = pl.pallas_call(
        _out_body,
        grid=(N // _RB,),
        in_specs=[
            pl.BlockSpec((NC, _RB, D), lambda i: (0, i, 0)),
            pl.BlockSpec((_RB, D), lambda i: (i, 0)),
            pl.BlockSpec((NC, _RB, WD), lambda i: (0, i, 0)),
            pl.BlockSpec((1, D), lambda i: (0, 0)),
        ],
        out_specs=pl.BlockSpec((_RB, D), lambda i: (i, 0)),
        out_shape=jax.ShapeDtypeStruct((N, D), jnp.float32),
    )(parts, hs, dega, b.reshape(1, D))
    return out
